# Initial kernel scaffold; baseline (speedup 1.0000x reference)
import jax
import jax.numpy as jnp
from jax.experimental import pallas as pl


def kernel(inputs):
    raise NotImplementedError("write your pallas kernel here")


# speedup vs baseline: 1.6169x; 1.6169x over previous
import jax
import jax.numpy as jnp
from jax import lax
from jax.experimental import pallas as pl
from jax.experimental.pallas import tpu as pltpu
from jax.experimental.pallas import tpu_sc as plsc

_V = 100000
_K = 16
_L = 16
_THRESH = 10


def _make_body(nc, nw, gpt, last):
    full_words = gpt * _L * _K
    last_words = last * _L * _K

    def body(in_hbm, out_hbm, rows_v, out_v):
        c = lax.axis_index("c")
        s = lax.axis_index("s")
        wid = s * nc + c
        is_last = wid == nw - 1
        in_base = wid * full_words
        out_base = wid * (gpt * _L)

        @pl.when(jnp.logical_not(is_last))
        def _():
            pltpu.sync_copy(in_hbm.at[pl.ds(in_base, full_words)], rows_v)

        @pl.when(is_last)
        def _():
            pltpu.sync_copy(in_hbm.at[pl.ds(in_base, last_words)],
                            rows_v.at[pl.ds(0, last_words)])

        lane_base = lax.iota(jnp.int32, _L) * _K
        ones = jnp.full((_L,), 1, jnp.int32)
        neg1 = jnp.full((_L,), -1, jnp.int32)

        def group(g, carry):
            base = g * (_L * _K) + lane_base
            xs = [plsc.load_gather(rows_v, [base + k]) for k in range(_K)]
            cand = xs[0]
            cnt = ones
            for k in range(1, _K):
                xk = xs[k]
                eq = xk == cand
                dead = cnt == 0
                cand = jnp.where(dead, xk, cand)
                cnt = jnp.where(dead, ones,
                                jnp.where(eq, cnt + 1, cnt - 1))
            tot = (xs[0] == cand).astype(jnp.int32)
            for k in range(1, _K):
                tot = tot + (xs[k] == cand).astype(jnp.int32)
            res = jnp.where(tot >= _THRESH, cand, neg1)
            out_v[pl.ds(g * _L, _L)] = res
            return carry

        lax.fori_loop(0, gpt, group, 0)

        @pl.when(jnp.logical_not(is_last))
        def _():
            pltpu.sync_copy(out_v, out_hbm.at[pl.ds(out_base, gpt * _L)])

        @pl.when(is_last)
        def _():
            pltpu.sync_copy(out_v.at[pl.ds(0, last * _L)],
                            out_hbm.at[pl.ds(out_base, last * _L)])

    return body


def kernel(inputs):
    info = plsc.get_sparse_core_info()
    nc, ns = info.num_cores, info.num_subcores
    nw = nc * ns
    groups = _V // _L
    gpt = -(-groups // nw)
    last = groups - (nw - 1) * gpt

    body = _make_body(nc, nw, gpt, last)
    mesh = plsc.VectorSubcoreMesh(core_axis_name="c", subcore_axis_name="s")
    out = pl.kernel(
        body,
        out_type=jax.ShapeDtypeStruct((_V,), jnp.int32),
        mesh=mesh,
        scratch_types=[
            pltpu.VMEM((gpt * _L * _K,), jnp.int32),
            pltpu.VMEM((gpt * _L,), jnp.int32),
        ],
        compiler_params=pltpu.CompilerParams(needs_layout_passes=False),
    )(inputs.reshape(-1))
    return out.reshape(_V, 1)


# speedup vs baseline: 1.6238x; 1.0043x over previous
import jax
import jax.numpy as jnp
from jax import lax
from jax.experimental import pallas as pl
from jax.experimental.pallas import tpu as pltpu
from jax.experimental.pallas import tpu_sc as plsc

_V = 100000
_K = 16
_L = 16
_THRESH = 10


def _make_body(nc, nw, gpt, last):
    full_words = gpt * _L * _K
    last_words = last * _L * _K

    def body(in_hbm, out_hbm, rows_v, out_v):
        c = lax.axis_index("c")
        s = lax.axis_index("s")
        wid = s * nc + c
        is_last = wid == nw - 1
        in_base = wid * full_words
        out_base = wid * (gpt * _L)

        @pl.when(jnp.logical_not(is_last))
        def _():
            pltpu.sync_copy(in_hbm.at[pl.ds(in_base, full_words)], rows_v)

        @pl.when(is_last)
        def _():
            pltpu.sync_copy(in_hbm.at[pl.ds(in_base, last_words)],
                            rows_v.at[pl.ds(0, last_words)])

        lane_base = lax.iota(jnp.int32, _L) * _K
        ones = jnp.full((_L,), 1, jnp.int32)
        neg1 = jnp.full((_L,), -1, jnp.int32)

        @plsc.parallel_loop(0, gpt, unroll=4)
        def _group(g):
            base = g * (_L * _K) + lane_base
            xs = [plsc.load_gather(rows_v, [base + k]) for k in range(_K)]
            cand = xs[0]
            cnt = ones
            for k in range(1, _K):
                xk = xs[k]
                eq = xk == cand
                dead = cnt == 0
                delta = jnp.where(eq, ones, neg1)
                cnt2 = cnt + delta
                cand = jnp.where(dead, xk, cand)
                cnt = jnp.where(dead, ones, cnt2)
            eqs = [(xs[k] == cand).astype(jnp.int32) for k in range(_K)]
            while len(eqs) > 1:
                eqs = [a + b for a, b in zip(eqs[::2], eqs[1::2])]
            res = jnp.where(eqs[0] >= _THRESH, cand, neg1)
            out_v[pl.ds(g * _L, _L)] = res

        @pl.when(jnp.logical_not(is_last))
        def _():
            pltpu.sync_copy(out_v, out_hbm.at[pl.ds(out_base, gpt * _L)])

        @pl.when(is_last)
        def _():
            pltpu.sync_copy(out_v.at[pl.ds(0, last * _L)],
                            out_hbm.at[pl.ds(out_base, last * _L)])

    return body


def kernel(inputs):
    info = plsc.get_sparse_core_info()
    nc, ns = info.num_cores, info.num_subcores
    nw = nc * ns
    groups = _V // _L
    gpt = -(-groups // nw)
    last = groups - (nw - 1) * gpt

    body = _make_body(nc, nw, gpt, last)
    mesh = plsc.VectorSubcoreMesh(core_axis_name="c", subcore_axis_name="s")
    out = pl.kernel(
        body,
        out_type=jax.ShapeDtypeStruct((_V,), jnp.int32),
        mesh=mesh,
        scratch_types=[
            pltpu.VMEM((gpt * _L * _K,), jnp.int32),
            pltpu.VMEM((gpt * _L,), jnp.int32),
        ],
        compiler_params=pltpu.CompilerParams(needs_layout_passes=False),
    )(inputs.reshape(-1))
    return out.reshape(_V, 1)


# speedup vs baseline: 4.3221x; 2.6618x over previous
import jax
import jax.numpy as jnp
from jax import lax
from jax.experimental import pallas as pl
from jax.experimental.pallas import tpu as pltpu
from jax.experimental.pallas import tpu_sc as plsc

_V = 100000
_K = 16
_L = 16
_THRESH = 10

_TCOLS = _V // 128
_TAIL = _V - _TCOLS * 128
_VPAD = (_TCOLS + 1) * 128


def _bm_select(xs, ones, neg1):
    cand = xs[0]
    cnt = ones
    for k in range(1, _K):
        xk = xs[k]
        eq = xk == cand
        dead = cnt == 0
        delta = jnp.where(eq, ones, neg1)
        cnt2 = cnt + delta
        cand = jnp.where(dead, xk, cand)
        cnt = jnp.where(dead, ones, cnt2)
    eqs = [(xs[k] == cand).astype(jnp.int32) for k in range(_K)]
    while len(eqs) > 1:
        eqs = [a + b for a, b in zip(eqs[::2], eqs[1::2])]
    return jnp.where(eqs[0] >= _THRESH, cand, neg1)


def _make_body(nc, nw):
    q, r = divmod(_TCOLS, nw)
    big_w, small_w = (q + 1) * 128, q * 128
    big_g, small_g = (q + 1) * 8, q * 8

    def body(in_hbm, tail_hbm, out_hbm, buf, out_v, tail_buf, tail_out):
        c = lax.axis_index("c")
        s = lax.axis_index("s")
        wid = s * nc + c
        is_big = wid < r
        col_base = jnp.where(is_big, wid * big_w,
                             r * big_w + (wid - r) * small_w)

        @pl.when(is_big)
        def _():
            pltpu.sync_copy(in_hbm.at[:, pl.ds(col_base, big_w)], buf)

        @pl.when(jnp.logical_not(is_big))
        def _():
            pltpu.sync_copy(in_hbm.at[:, pl.ds(col_base, small_w)],
                            buf.at[:, pl.ds(0, small_w)])

        ones = jnp.full((_L,), 1, jnp.int32)
        neg1 = jnp.full((_L,), -1, jnp.int32)

        @plsc.parallel_loop(0, big_g, unroll=4)
        def _group(g):
            xs = [buf[k, pl.ds(g * _L, _L)] for k in range(_K)]
            out_v[0, pl.ds(g * _L, _L)] = _bm_select(xs, ones, neg1)

        @pl.when(is_big)
        def _():
            pltpu.sync_copy(out_v, out_hbm.at[:, pl.ds(col_base, big_w)])

        @pl.when(jnp.logical_not(is_big))
        def _():
            pltpu.sync_copy(out_v.at[:, pl.ds(0, small_w)],
                            out_hbm.at[:, pl.ds(col_base, small_w)])

        @pl.when(wid == nw - 1)
        def _():
            pltpu.sync_copy(tail_hbm, tail_buf)
            for g in range(_TAIL // _L):
                xs = [tail_buf[k, pl.ds(g * _L, _L)] for k in range(_K)]
                tail_out[0, pl.ds(g * _L, _L)] = _bm_select(xs, ones, neg1)
            pltpu.sync_copy(tail_out, out_hbm.at[:, pl.ds(_TCOLS * 128, 128)])

    return body


def kernel(inputs):
    info = plsc.get_sparse_core_info()
    nc, ns = info.num_cores, info.num_subcores
    nw = nc * ns
    q, r = divmod(_TCOLS, nw)
    big_w = (q + 1) * 128

    body = _make_body(nc, nw)
    mesh = plsc.VectorSubcoreMesh(core_axis_name="c", subcore_axis_name="s")
    xt = inputs.T
    tail = xt[:, _TCOLS * 128:]
    out = pl.kernel(
        body,
        out_type=jax.ShapeDtypeStruct((1, _VPAD), jnp.int32),
        mesh=mesh,
        scratch_types=[
            pltpu.VMEM((_K, big_w), jnp.int32),
            pltpu.VMEM((1, big_w), jnp.int32),
            pltpu.VMEM((_K, _TAIL), jnp.int32),
            pltpu.VMEM((1, 128), jnp.int32),
        ],
        compiler_params=pltpu.CompilerParams(
            use_tc_tiling_on_sc=True,
            needs_layout_passes=False,
        ),
    )(xt, tail)
    return out[0, :_V].reshape(_V, 1)
